# Initial kernel scaffold; baseline (speedup 1.0000x reference)
import jax
import jax.numpy as jnp
from jax.experimental import pallas as pl


def kernel(mem, input_features, labels, current_flags):
    raise NotImplementedError("write your pallas kernel here")


# speedup vs baseline: 21.7913x; 21.7913x over previous
import jax
import jax.numpy as jnp
from jax import lax
from jax.experimental import pallas as pl
from jax.experimental.pallas import tpu as pltpu
from jax.experimental.pallas import tpu_sc as plsc


def _index_body(lab_col_ref, lab_row_ref, flags_ref, gidx_ref, srow_ref):
    b = lab_col_ref.shape[0]
    n_labels = flags_ref.shape[1]
    q = 16

    lab_col = lab_col_ref[...]
    lab_row = lab_row_ref[...]
    flags = flags_ref[...]

    ii = lax.broadcasted_iota(jnp.int32, (b, b), 0)
    jj = lax.broadcasted_iota(jnp.int32, (b, b), 1)
    same = lab_col == lab_row

    rank_col = jnp.sum(jnp.where(same & (jj < ii), 1, 0),
                       axis=1, keepdims=True)
    rank_row = jnp.sum(jnp.where(same & (ii < jj), 1, 0),
                       axis=0, keepdims=True)

    ff = lax.broadcasted_iota(jnp.int32, (b, n_labels), 1)
    fl_col = jnp.sum(jnp.where(ff == lab_col, flags, 0),
                     axis=1, keepdims=True)

    pos_col = (fl_col + rank_col) % q
    srow_ref[...] = lab_col * q + pos_col

    slot_eq = same & ((rank_col % q) == (rank_row % q))
    gidx_ref[...] = jnp.max(jnp.where(slot_eq, jj, -1), axis=1, keepdims=True)


def _compute_indices(labels, current_flags):
    b = labels.shape[0]
    out_t = [
        jax.ShapeDtypeStruct((b, 1), jnp.int32),
        jax.ShapeDtypeStruct((b, 1), jnp.int32),
    ]
    gidx, srow = pl.pallas_call(_index_body, out_shape=out_t)(
        labels.reshape(b, 1),
        labels.reshape(1, b),
        current_flags.reshape(1, -1),
    )
    return gidx.reshape(b), srow.reshape(b)


def _make_sc_scatter(b, d, n_rows):
    info = plsc.get_sparse_core_info()
    nc, ns = info.num_cores, info.num_subcores
    nw = nc * ns
    per_w = b // nw
    assert per_w * nw == b and per_w % 8 == 0

    mesh = plsc.VectorSubcoreMesh(core_axis_name="c", subcore_axis_name="s")

    @pl.kernel(
        mesh=mesh,
        out_type=(),
        scratch_types=[
            pltpu.VMEM((per_w,), jnp.int32),
            pltpu.VMEM((per_w,), jnp.int32),
            pltpu.VMEM((per_w, d), jnp.float32),
            pltpu.SemaphoreType.DMA,
            pltpu.SemaphoreType.DMA,
        ],
    )
    def sc_scatter(mem_ref, feat_hbm, gidx_hbm, srow_hbm,
                   gidx_v, srow_v, rows_v, sem_g, sem_s):
        wid = lax.axis_index("s") * nc + lax.axis_index("c")
        base = wid * per_w
        pltpu.sync_copy(gidx_hbm.at[pl.ds(base, per_w)], gidx_v)
        pltpu.sync_copy(srow_hbm.at[pl.ds(base, per_w)], srow_v)
        pltpu.async_copy(feat_hbm.at[gidx_v], rows_v, sem_g).wait()
        pltpu.async_copy(rows_v, mem_ref.at[srow_v], sem_s).wait()

    return sc_scatter


def kernel(mem, input_features, labels, current_flags):
    n_labels, q, d = mem.shape
    b = input_features.shape[0]

    gidx, srow = _compute_indices(labels, current_flags)

    mem_ref = jax.new_ref(mem.reshape(n_labels * q, d))
    sc = _make_sc_scatter(b, d, n_labels * q)
    sc(mem_ref, input_features, gidx, srow)
    return mem_ref[...].reshape(n_labels, q, d)


# speedup vs baseline: 21.8471x; 1.0026x over previous
import jax
import jax.numpy as jnp
from jax import lax
from jax.experimental import pallas as pl
from jax.experimental.pallas import tpu as pltpu
from jax.experimental.pallas import tpu_sc as plsc


def _index_body(lab_col_ref, lab_row_ref, flags_ref, gidx_ref, srow_ref):
    b = lab_col_ref.shape[0]
    n_labels = flags_ref.shape[1]
    q = 16

    lab_col = lab_col_ref[...]
    lab_row = lab_row_ref[...]
    flags = flags_ref[...]

    ii = lax.broadcasted_iota(jnp.int32, (b, b), 0)
    jj = lax.broadcasted_iota(jnp.int32, (b, b), 1)
    same = lab_col == lab_row

    rank_col = jnp.sum(jnp.where(same & (jj < ii), 1, 0),
                       axis=1, keepdims=True)
    rank_row = jnp.sum(jnp.where(same & (ii < jj), 1, 0),
                       axis=0, keepdims=True)

    ff = lax.broadcasted_iota(jnp.int32, (b, n_labels), 1)
    fl_col = jnp.sum(jnp.where(ff == lab_col, flags, 0),
                     axis=1, keepdims=True)

    pos_col = (fl_col + rank_col) % q
    srow_ref[...] = lab_col * q + pos_col

    slot_eq = same & ((rank_col % q) == (rank_row % q))
    gidx_ref[...] = jnp.max(jnp.where(slot_eq, jj, -1), axis=1, keepdims=True)


def _compute_indices(labels, current_flags):
    b = labels.shape[0]
    out_t = [
        jax.ShapeDtypeStruct((b, 1), jnp.int32),
        jax.ShapeDtypeStruct((b, 1), jnp.int32),
    ]
    gidx, srow = pl.pallas_call(_index_body, out_shape=out_t)(
        labels.reshape(b, 1),
        labels.reshape(1, b),
        current_flags.reshape(1, -1),
    )
    return gidx.reshape(b), srow.reshape(b)


def _make_sc_scatter(b, d, n_rows):
    info = plsc.get_sparse_core_info()
    nc, ns = info.num_cores, info.num_subcores
    nw = nc * ns
    per_w = b // nw
    assert per_w * nw == b and per_w % 8 == 0

    mesh = plsc.VectorSubcoreMesh(core_axis_name="c", subcore_axis_name="s")

    @pl.kernel(
        mesh=mesh,
        out_type=(),
        scratch_types=[
            pltpu.VMEM((per_w,), jnp.int32),
            pltpu.VMEM((per_w,), jnp.int32),
            pltpu.VMEM((per_w, d), jnp.float32),
            pltpu.SemaphoreType.DMA,
            pltpu.SemaphoreType.DMA,
            pltpu.SemaphoreType.DMA,
            pltpu.SemaphoreType.DMA,
        ],
    )
    def sc_scatter(mem_ref, feat_hbm, gidx_hbm, srow_hbm,
                   gidx_v, srow_v, rows_v, sem_i1, sem_i2, sem_g, sem_s):
        wid = lax.axis_index("s") * nc + lax.axis_index("c")
        base = wid * per_w
        ld_g = pltpu.async_copy(gidx_hbm.at[pl.ds(base, per_w)], gidx_v, sem_i1)
        ld_s = pltpu.async_copy(srow_hbm.at[pl.ds(base, per_w)], srow_v, sem_i2)
        ld_g.wait()
        gather = pltpu.async_copy(feat_hbm.at[gidx_v], rows_v, sem_g)
        ld_s.wait()
        gather.wait()
        pltpu.async_copy(rows_v, mem_ref.at[srow_v], sem_s).wait()

    return sc_scatter


def kernel(mem, input_features, labels, current_flags):
    n_labels, q, d = mem.shape
    b = input_features.shape[0]

    gidx, srow = _compute_indices(labels, current_flags)

    mem_ref = jax.new_ref(mem.reshape(n_labels * q, d))
    sc = _make_sc_scatter(b, d, n_labels * q)
    sc(mem_ref, input_features, gidx, srow)
    return mem_ref[...].reshape(n_labels, q, d)
